# baseline (speedup 1.0000x reference)
import functools

import jax
import jax.numpy as jnp
from jax.experimental import pallas as pl
from jax.experimental.pallas import tpu as pltpu
from jax.experimental.pallas import tpu_sc as plsc

N = 4096
D = 768
E = 16
K = 2
T = 256
R = 12288
NT = R // T
TB = 512

NC, NS = 2, 16
NW = NC * NS
CH = 128


def _router_body(x_ref, rw_ref, g_ref, i_ref):
    logits = jnp.dot(x_ref[...], rw_ref[...], preferred_element_type=jnp.float32)
    iota = jax.lax.broadcasted_iota(jnp.int32, logits.shape, 1)
    m1 = jnp.max(logits, axis=1, keepdims=True)
    i1 = jnp.min(jnp.where(logits == m1, iota, E), axis=1, keepdims=True)
    masked = jnp.where(iota == i1, jnp.finfo(jnp.float32).min, logits)
    m2 = jnp.max(masked, axis=1, keepdims=True)
    i2 = jnp.min(jnp.where(masked == m2, iota, E), axis=1, keepdims=True)
    e2 = jnp.exp(m2 - m1)
    g1 = 1.0 / (1.0 + e2)
    g_ref[...] = jnp.concatenate([g1, 1.0 - g1], axis=1)
    i_ref[...] = jnp.concatenate([i1, i2], axis=1).astype(jnp.int32)


def _router(x_flat, router_w):
    return pl.pallas_call(
        _router_body,
        grid=(N // TB,),
        in_specs=[
            pl.BlockSpec((TB, D), lambda i: (i, 0)),
            pl.BlockSpec((D, E), lambda i: (0, 0)),
        ],
        out_specs=[
            pl.BlockSpec((TB, K), lambda i: (i, 0)),
            pl.BlockSpec((TB, K), lambda i: (i, 0)),
        ],
        out_shape=[
            jax.ShapeDtypeStruct((N, K), jnp.float32),
            jax.ShapeDtypeStruct((N, K), jnp.int32),
        ],
    )(x_flat, router_w)


def _dispatch_indices(eidx):
    ef = eidx.reshape(-1)
    oh = (ef[:, None] == jnp.arange(E, dtype=jnp.int32)[None, :]).astype(jnp.int32)
    cum = jnp.cumsum(oh, axis=0)
    rank = jnp.take_along_axis(cum, ef[:, None], axis=1)[:, 0] - 1
    counts = cum[-1]
    padded = ((counts + T - 1) // T) * T
    offsets = jnp.concatenate(
        [jnp.zeros(1, jnp.int32), jnp.cumsum(padded)[:-1].astype(jnp.int32)])
    pos = offsets[ef] + rank
    src = jnp.zeros(R, jnp.int32).at[pos].set(
        jnp.arange(K * N, dtype=jnp.int32) // K)
    tile_start = (jnp.arange(NT, dtype=jnp.int32) * T)[:, None]
    tile_expert = (jnp.sum(offsets[None, :] <= tile_start, axis=1) - 1).astype(jnp.int32)
    return src, tile_expert, pos


def _sc_gather(table, idx, nrows):
    b_per_w = nrows // NW
    mesh = plsc.VectorSubcoreMesh(core_axis_name="c", subcore_axis_name="s")

    @functools.partial(
        pl.kernel,
        out_type=jax.ShapeDtypeStruct((nrows, D), table.dtype),
        mesh=mesh,
        scratch_types=[
            pltpu.VMEM((CH,), jnp.int32),
            pltpu.VMEM((CH, D), table.dtype),
            pltpu.SemaphoreType.DMA,
        ],
    )
    def gk(table_hbm, idx_hbm, out_hbm, idx_v, rows_v, sem):
        wid = jax.lax.axis_index("s") * NC + jax.lax.axis_index("c")
        base = wid * b_per_w

        @pl.loop(0, b_per_w, step=CH)
        def _(off):
            pltpu.sync_copy(idx_hbm.at[pl.ds(base + off, CH)], idx_v)
            pltpu.async_copy(table_hbm.at[idx_v], rows_v, sem).wait()
            pltpu.sync_copy(rows_v, out_hbm.at[pl.ds(base + off, CH)])

    return gk(table, idx)


def _mm_body(te_ref, xs_ref, w_ref, b_ref, o_ref):
    acc = jax.lax.dot_general(
        xs_ref[...].astype(jnp.bfloat16), w_ref[0], (((1,), (1,)), ((), ())),
        preferred_element_type=jnp.float32)
    o_ref[...] = acc + b_ref[0]


def _grouped_mm(xs, tile_expert, expert_w, expert_b):
    grid_spec = pltpu.PrefetchScalarGridSpec(
        num_scalar_prefetch=1,
        grid=(NT,),
        in_specs=[
            pl.BlockSpec((T, D), lambda i, te: (i, 0)),
            pl.BlockSpec((1, D, D), lambda i, te: (te[i], 0, 0)),
            pl.BlockSpec((1, 1, D), lambda i, te: (te[i], 0, 0)),
        ],
        out_specs=pl.BlockSpec((T, D), lambda i, te: (i, 0)),
    )
    return pl.pallas_call(
        _mm_body,
        grid_spec=grid_spec,
        out_shape=jax.ShapeDtypeStruct((R, D), jnp.float32),
    )(tile_expert, xs, expert_w.astype(jnp.bfloat16), expert_b.reshape(E, 1, D))


def _comb_body(y_ref, g_ref, o_ref):
    g = g_ref[...]
    y = y_ref[...]
    o_ref[...] = y[:, 0, :] * g[:, 0:1] + y[:, 1, :] * g[:, 1:2]


def _combine(yg3, gates):
    return pl.pallas_call(
        _comb_body,
        grid=(N // TB,),
        in_specs=[
            pl.BlockSpec((TB, K, D), lambda i: (i, 0, 0)),
            pl.BlockSpec((TB, K), lambda i: (i, 0)),
        ],
        out_specs=pl.BlockSpec((TB, D), lambda i: (i, 0)),
        out_shape=jax.ShapeDtypeStruct((N, D), jnp.float32),
    )(yg3, gates)


def kernel(x, router_w, expert_w, expert_b):
    b, s, d = x.shape
    x_flat = x.reshape(N, D)
    gates, eidx = _router(x_flat, router_w)
    src, tile_expert, pos = _dispatch_indices(eidx)
    xs = _sc_gather(x_flat, src, R)
    ys = _grouped_mm(xs, tile_expert, expert_w, expert_b)
    yg = _sc_gather(ys, pos, K * N)
    out = _combine(yg.reshape(N, K, D), gates)
    return out.reshape(b, s, D)


# speedup vs baseline: 1.7481x; 1.7481x over previous
import functools

import jax
import jax.numpy as jnp
from jax.experimental import pallas as pl
from jax.experimental.pallas import tpu as pltpu
from jax.experimental.pallas import tpu_sc as plsc

N = 4096
D = 768
E = 16
K = 2
T = 256
R = 12288
NT = R // T
TB = 512

NC, NS = 2, 16
NW = NC * NS
CH = 128


def _router_body(x_ref, rw_ref, g_ref, i_ref):
    logits = jnp.dot(x_ref[...], rw_ref[...], preferred_element_type=jnp.float32)
    iota = jax.lax.broadcasted_iota(jnp.int32, logits.shape, 1)
    m1 = jnp.max(logits, axis=1, keepdims=True)
    i1 = jnp.min(jnp.where(logits == m1, iota, E), axis=1, keepdims=True)
    masked = jnp.where(iota == i1, jnp.finfo(jnp.float32).min, logits)
    m2 = jnp.max(masked, axis=1, keepdims=True)
    i2 = jnp.min(jnp.where(masked == m2, iota, E), axis=1, keepdims=True)
    e2 = jnp.exp(m2 - m1)
    g1 = 1.0 / (1.0 + e2)
    g_ref[...] = jnp.concatenate([g1, 1.0 - g1], axis=1)
    i_ref[...] = jnp.concatenate([i1, i2], axis=1).astype(jnp.int32)


def _router(x_flat, router_w):
    return pl.pallas_call(
        _router_body,
        grid=(N // TB,),
        in_specs=[
            pl.BlockSpec((TB, D), lambda i: (i, 0)),
            pl.BlockSpec((D, E), lambda i: (0, 0)),
        ],
        out_specs=[
            pl.BlockSpec((TB, K), lambda i: (i, 0)),
            pl.BlockSpec((TB, K), lambda i: (i, 0)),
        ],
        out_shape=[
            jax.ShapeDtypeStruct((N, K), jnp.float32),
            jax.ShapeDtypeStruct((N, K), jnp.int32),
        ],
    )(x_flat, router_w)


def _dispatch_indices(eidx):
    ef = eidx.reshape(-1)
    oh = (ef[:, None] == jnp.arange(E, dtype=jnp.int32)[None, :]).astype(jnp.int32)
    cum = jnp.cumsum(oh, axis=0)
    rank = jnp.sum(oh * cum, axis=1) - 1
    counts = cum[-1]
    padded = ((counts + T - 1) // T) * T
    offsets = jnp.concatenate(
        [jnp.zeros(1, jnp.int32), jnp.cumsum(padded)[:-1].astype(jnp.int32)])
    pos = jnp.sum(oh * offsets[None, :], axis=1) + rank
    src = (jnp.arange(R, dtype=jnp.int32) % N).at[pos].set(
        jnp.arange(K * N, dtype=jnp.int32) // K)
    tile_start = (jnp.arange(NT, dtype=jnp.int32) * T)[:, None]
    tile_expert = (jnp.sum(offsets[None, :] <= tile_start, axis=1) - 1).astype(jnp.int32)
    return src, tile_expert, pos


def _sc_gather(table, idx, nrows):
    b_per_w = nrows // NW
    mesh = plsc.VectorSubcoreMesh(core_axis_name="c", subcore_axis_name="s")

    @functools.partial(
        pl.kernel,
        out_type=jax.ShapeDtypeStruct((nrows, D), table.dtype),
        mesh=mesh,
        scratch_types=[
            pltpu.VMEM((CH,), jnp.int32),
            pltpu.VMEM((CH, D), table.dtype),
            pltpu.SemaphoreType.DMA,
        ],
    )
    def gk(table_hbm, idx_hbm, out_hbm, idx_v, rows_v, sem):
        wid = jax.lax.axis_index("s") * NC + jax.lax.axis_index("c")
        base = wid * b_per_w

        @pl.loop(0, b_per_w, step=CH)
        def _(off):
            pltpu.sync_copy(idx_hbm.at[pl.ds(base + off, CH)], idx_v)
            pltpu.async_copy(table_hbm.at[idx_v], rows_v, sem).wait()
            pltpu.sync_copy(rows_v, out_hbm.at[pl.ds(base + off, CH)])

    return gk(table, idx)


def _mm_body(te_ref, xs_ref, w_ref, b_ref, o_ref):
    acc = jax.lax.dot_general(
        xs_ref[...].astype(jnp.bfloat16), w_ref[0], (((1,), (1,)), ((), ())),
        preferred_element_type=jnp.float32)
    o_ref[...] = acc + b_ref[0]


def _grouped_mm(xs, tile_expert, expert_w, expert_b):
    grid_spec = pltpu.PrefetchScalarGridSpec(
        num_scalar_prefetch=1,
        grid=(NT,),
        in_specs=[
            pl.BlockSpec((T, D), lambda i, te: (i, 0)),
            pl.BlockSpec((1, D, D), lambda i, te: (te[i], 0, 0)),
            pl.BlockSpec((1, 1, D), lambda i, te: (te[i], 0, 0)),
        ],
        out_specs=pl.BlockSpec((T, D), lambda i, te: (i, 0)),
    )
    return pl.pallas_call(
        _mm_body,
        grid_spec=grid_spec,
        out_shape=jax.ShapeDtypeStruct((R, D), jnp.float32),
    )(tile_expert, xs, expert_w.astype(jnp.bfloat16), expert_b.reshape(E, 1, D))


def _comb_body(y_ref, g_ref, o_ref):
    g = g_ref[...]
    y = y_ref[...]
    o_ref[...] = y[:, 0, :] * g[:, 0:1] + y[:, 1, :] * g[:, 1:2]


def _combine(yg3, gates):
    return pl.pallas_call(
        _comb_body,
        grid=(N // TB,),
        in_specs=[
            pl.BlockSpec((TB, K, D), lambda i: (i, 0, 0)),
            pl.BlockSpec((TB, K), lambda i: (i, 0)),
        ],
        out_specs=pl.BlockSpec((TB, D), lambda i: (i, 0)),
        out_shape=jax.ShapeDtypeStruct((N, D), jnp.float32),
    )(yg3, gates)


def kernel(x, router_w, expert_w, expert_b):
    b, s, d = x.shape
    x_flat = x.reshape(N, D)
    gates, eidx = _router(x_flat, router_w)
    src, tile_expert, pos = _dispatch_indices(eidx)
    xs = _sc_gather(x_flat, src, R)
    ys = _grouped_mm(xs, tile_expert, expert_w, expert_b)
    yg = _sc_gather(ys, pos, K * N)
    out = _combine(yg.reshape(N, K, D), gates)
    return out.reshape(b, s, D)


# speedup vs baseline: 2.4669x; 1.4112x over previous
import functools

import jax
import jax.numpy as jnp
from jax.experimental import pallas as pl
from jax.experimental.pallas import tpu as pltpu
from jax.experimental.pallas import tpu_sc as plsc

N = 4096
D = 768
E = 16
K = 2
T = 256
R = 12288
NT = R // T
TB = 512

NC, NS = 2, 16
NW = NC * NS
CH = 128


def _router_body(x_ref, rw_ref, g_ref, i_ref):
    logits = jnp.dot(x_ref[...], rw_ref[...], preferred_element_type=jnp.float32)
    iota = jax.lax.broadcasted_iota(jnp.int32, logits.shape, 1)
    m1 = jnp.max(logits, axis=1, keepdims=True)
    i1 = jnp.min(jnp.where(logits == m1, iota, E), axis=1, keepdims=True)
    masked = jnp.where(iota == i1, jnp.finfo(jnp.float32).min, logits)
    m2 = jnp.max(masked, axis=1, keepdims=True)
    i2 = jnp.min(jnp.where(masked == m2, iota, E), axis=1, keepdims=True)
    e2 = jnp.exp(m2 - m1)
    g1 = 1.0 / (1.0 + e2)
    g_ref[...] = jnp.concatenate([g1, 1.0 - g1], axis=1)
    i_ref[...] = jnp.concatenate([i1, i2], axis=1).astype(jnp.int32)


def _router(x_flat, router_w):
    return pl.pallas_call(
        _router_body,
        grid=(N // TB,),
        in_specs=[
            pl.BlockSpec((TB, D), lambda i: (i, 0)),
            pl.BlockSpec((D, E), lambda i: (0, 0)),
        ],
        out_specs=[
            pl.BlockSpec((TB, K), lambda i: (i, 0)),
            pl.BlockSpec((TB, K), lambda i: (i, 0)),
        ],
        out_shape=[
            jax.ShapeDtypeStruct((N, K), jnp.float32),
            jax.ShapeDtypeStruct((N, K), jnp.int32),
        ],
    )(x_flat, router_w)


def _dispatch_indices(eidx):
    ef = eidx.reshape(-1)
    oh = (ef[:, None] == jnp.arange(E, dtype=jnp.int32)[None, :]).astype(jnp.int32)
    cum = jnp.cumsum(oh, axis=0)
    rank = jnp.sum(oh * cum, axis=1) - 1
    counts = cum[-1]
    padded = ((counts + T - 1) // T) * T
    offsets = jnp.concatenate(
        [jnp.zeros(1, jnp.int32), jnp.cumsum(padded)[:-1].astype(jnp.int32)])
    pos = jnp.sum(oh * offsets[None, :], axis=1) + rank
    src = (jnp.arange(R, dtype=jnp.int32) % N).at[pos].set(
        jnp.arange(K * N, dtype=jnp.int32) // K)
    tile_start = (jnp.arange(NT, dtype=jnp.int32) * T)[:, None]
    tile_expert = (jnp.sum(offsets[None, :] <= tile_start, axis=1) - 1).astype(jnp.int32)
    return src, tile_expert, pos


def _sc_gather(table, idx, nrows):
    b_per_w = nrows // NW
    mesh = plsc.VectorSubcoreMesh(core_axis_name="c", subcore_axis_name="s")

    @functools.partial(
        pl.kernel,
        out_type=jax.ShapeDtypeStruct((nrows, D), table.dtype),
        mesh=mesh,
        scratch_types=[
            pltpu.VMEM((CH,), jnp.int32),
            pltpu.VMEM((CH, D), table.dtype),
            pltpu.SemaphoreType.DMA,
        ],
    )
    def gk(table_hbm, idx_hbm, out_hbm, idx_v, rows_v, sem):
        wid = jax.lax.axis_index("s") * NC + jax.lax.axis_index("c")
        base = wid * b_per_w

        @pl.loop(0, b_per_w, step=CH)
        def _(off):
            pltpu.sync_copy(idx_hbm.at[pl.ds(base + off, CH)], idx_v)
            pltpu.async_copy(table_hbm.at[idx_v], rows_v, sem).wait()
            pltpu.sync_copy(rows_v, out_hbm.at[pl.ds(base + off, CH)])

    return gk(table, idx)


def _sc_gather2(table, idx0, idx1):
    b_per_w = N // NW
    mesh = plsc.VectorSubcoreMesh(core_axis_name="c", subcore_axis_name="s")

    @functools.partial(
        pl.kernel,
        out_type=[
            jax.ShapeDtypeStruct((N, D), table.dtype),
            jax.ShapeDtypeStruct((N, D), table.dtype),
        ],
        mesh=mesh,
        scratch_types=[
            pltpu.VMEM((CH,), jnp.int32),
            pltpu.VMEM((CH, D), table.dtype),
            pltpu.SemaphoreType.DMA,
        ],
    )
    def gk(table_hbm, i0_hbm, i1_hbm, o0_hbm, o1_hbm, idx_v, rows_v, sem):
        wid = jax.lax.axis_index("s") * NC + jax.lax.axis_index("c")
        base = wid * b_per_w

        @pl.loop(0, b_per_w, step=CH)
        def _(off):
            pltpu.sync_copy(i0_hbm.at[pl.ds(base + off, CH)], idx_v)
            pltpu.async_copy(table_hbm.at[idx_v], rows_v, sem).wait()
            pltpu.sync_copy(rows_v, o0_hbm.at[pl.ds(base + off, CH)])
            pltpu.sync_copy(i1_hbm.at[pl.ds(base + off, CH)], idx_v)
            pltpu.async_copy(table_hbm.at[idx_v], rows_v, sem).wait()
            pltpu.sync_copy(rows_v, o1_hbm.at[pl.ds(base + off, CH)])

    return gk(table, idx0, idx1)


def _mm_body(te_ref, xs_ref, w_ref, b_ref, o_ref):
    acc = jax.lax.dot_general(
        xs_ref[...], w_ref[0], (((1,), (1,)), ((), ())),
        preferred_element_type=jnp.float32)
    o_ref[...] = acc + b_ref[0]


def _grouped_mm(xs, tile_expert, expert_w, expert_b):
    grid_spec = pltpu.PrefetchScalarGridSpec(
        num_scalar_prefetch=1,
        grid=(NT,),
        in_specs=[
            pl.BlockSpec((T, D), lambda i, te: (i, 0)),
            pl.BlockSpec((1, D, D), lambda i, te: (te[i], 0, 0)),
            pl.BlockSpec((1, 1, D), lambda i, te: (te[i], 0, 0)),
        ],
        out_specs=pl.BlockSpec((T, D), lambda i, te: (i, 0)),
    )
    return pl.pallas_call(
        _mm_body,
        grid_spec=grid_spec,
        out_shape=jax.ShapeDtypeStruct((R, D), jnp.float32),
    )(tile_expert, xs, expert_w, expert_b.reshape(E, 1, D))


def _comb_body(y0_ref, y1_ref, g_ref, o_ref):
    g = g_ref[...]
    o_ref[...] = y0_ref[...] * g[:, 0:1] + y1_ref[...] * g[:, 1:2]


def _combine(y0, y1, gates):
    return pl.pallas_call(
        _comb_body,
        grid=(N // TB,),
        in_specs=[
            pl.BlockSpec((TB, D), lambda i: (i, 0)),
            pl.BlockSpec((TB, D), lambda i: (i, 0)),
            pl.BlockSpec((TB, K), lambda i: (i, 0)),
        ],
        out_specs=pl.BlockSpec((TB, D), lambda i: (i, 0)),
        out_shape=jax.ShapeDtypeStruct((N, D), jnp.float32),
    )(y0, y1, gates)


def kernel(x, router_w, expert_w, expert_b):
    b, s, d = x.shape
    x_flat = x.reshape(N, D)
    gates, eidx = _router(x_flat, router_w)
    src, tile_expert, pos = _dispatch_indices(eidx)
    xs = _sc_gather(x_flat, src, R)
    ys = _grouped_mm(xs, tile_expert, expert_w, expert_b)
    pos2 = pos.reshape(N, K)
    y0, y1 = _sc_gather2(ys, pos2[:, 0], pos2[:, 1])
    out = _combine(y0, y1, gates)
    return out.reshape(b, s, D)


# speedup vs baseline: 3.0648x; 1.2424x over previous
import functools

import jax
import jax.numpy as jnp
from jax.experimental import pallas as pl
from jax.experimental.pallas import tpu as pltpu
from jax.experimental.pallas import tpu_sc as plsc

N = 4096
D = 768
E = 16
K = 2
T = 256
R = 12288
NT = R // T
TB = 512

NC, NS = 2, 16
NW = NC * NS
CH = 128


def _router_body(x_ref, rw_ref, g_ref, i_ref):
    logits = jnp.dot(x_ref[...], rw_ref[...], preferred_element_type=jnp.float32)
    iota = jax.lax.broadcasted_iota(jnp.int32, logits.shape, 1)
    m1 = jnp.max(logits, axis=1, keepdims=True)
    i1 = jnp.min(jnp.where(logits == m1, iota, E), axis=1, keepdims=True)
    masked = jnp.where(iota == i1, jnp.finfo(jnp.float32).min, logits)
    m2 = jnp.max(masked, axis=1, keepdims=True)
    i2 = jnp.min(jnp.where(masked == m2, iota, E), axis=1, keepdims=True)
    e2 = jnp.exp(m2 - m1)
    g1 = 1.0 / (1.0 + e2)
    g_ref[...] = jnp.concatenate([g1, 1.0 - g1], axis=1)
    i_ref[...] = jnp.concatenate([i1, i2], axis=1).astype(jnp.int32)


def _router(x_flat, router_w):
    return pl.pallas_call(
        _router_body,
        grid=(N // TB,),
        in_specs=[
            pl.BlockSpec((TB, D), lambda i: (i, 0)),
            pl.BlockSpec((D, E), lambda i: (0, 0)),
        ],
        out_specs=[
            pl.BlockSpec((TB, K), lambda i: (i, 0)),
            pl.BlockSpec((TB, K), lambda i: (i, 0)),
        ],
        out_shape=[
            jax.ShapeDtypeStruct((N, K), jnp.float32),
            jax.ShapeDtypeStruct((N, K), jnp.int32),
        ],
    )(x_flat, router_w)


def _dispatch_indices(eidx):
    ef = eidx.reshape(-1)
    oh = (ef[:, None] == jnp.arange(E, dtype=jnp.int32)[None, :]).astype(jnp.int32)
    cum = jnp.cumsum(oh, axis=0)
    rank = jnp.sum(oh * cum, axis=1) - 1
    counts = cum[-1]
    padded = ((counts + T - 1) // T) * T
    offsets = jnp.concatenate(
        [jnp.zeros(1, jnp.int32), jnp.cumsum(padded)[:-1].astype(jnp.int32)])
    pos = jnp.sum(oh * offsets[None, :], axis=1) + rank
    tile_start = (jnp.arange(NT, dtype=jnp.int32) * T)[:, None]
    tile_expert = (jnp.sum(offsets[None, :] <= tile_start, axis=1) - 1).astype(jnp.int32)
    return tile_expert, pos


def _sc_dispatch(x_flat, idx0, idx1):
    b_per_w = N // NW
    mesh = plsc.VectorSubcoreMesh(core_axis_name="c", subcore_axis_name="s")

    @functools.partial(
        pl.kernel,
        out_type=jax.ShapeDtypeStruct((R, D), x_flat.dtype),
        mesh=mesh,
        scratch_types=[
            pltpu.VMEM((1, CH), jnp.int32),
            pltpu.VMEM((CH, D), x_flat.dtype),
            pltpu.SemaphoreType.DMA,
        ],
    )
    def sk(x_hbm, i0_hbm, i1_hbm, out_hbm, idx_v, rows_v, sem):
        wid = jax.lax.axis_index("s") * NC + jax.lax.axis_index("c")
        base = wid * b_per_w

        @pl.loop(0, b_per_w, step=CH)
        def _(off):
            pltpu.sync_copy(x_hbm.at[pl.ds(base + off, CH)], rows_v)
            pltpu.sync_copy(i0_hbm.at[pl.ds(0, 1), pl.ds(base + off, CH)], idx_v)
            pltpu.async_copy(rows_v, out_hbm.at[idx_v.at[0]], sem).wait()
            pltpu.sync_copy(i1_hbm.at[pl.ds(0, 1), pl.ds(base + off, CH)], idx_v)
            pltpu.async_copy(rows_v, out_hbm.at[idx_v.at[0]], sem).wait()

    return sk(x_flat, idx0.reshape(1, N), idx1.reshape(1, N))


def _sc_gather2(table, idx0, idx1):
    b_per_w = N // NW
    mesh = plsc.VectorSubcoreMesh(core_axis_name="c", subcore_axis_name="s")

    @functools.partial(
        pl.kernel,
        out_type=[
            jax.ShapeDtypeStruct((N, D), table.dtype),
            jax.ShapeDtypeStruct((N, D), table.dtype),
        ],
        mesh=mesh,
        scratch_types=[
            pltpu.VMEM((CH,), jnp.int32),
            pltpu.VMEM((CH, D), table.dtype),
            pltpu.SemaphoreType.DMA,
        ],
    )
    def gk(table_hbm, i0_hbm, i1_hbm, o0_hbm, o1_hbm, idx_v, rows_v, sem):
        wid = jax.lax.axis_index("s") * NC + jax.lax.axis_index("c")
        base = wid * b_per_w

        @pl.loop(0, b_per_w, step=CH)
        def _(off):
            pltpu.sync_copy(i0_hbm.at[pl.ds(base + off, CH)], idx_v)
            pltpu.async_copy(table_hbm.at[idx_v], rows_v, sem).wait()
            pltpu.sync_copy(rows_v, o0_hbm.at[pl.ds(base + off, CH)])
            pltpu.sync_copy(i1_hbm.at[pl.ds(base + off, CH)], idx_v)
            pltpu.async_copy(table_hbm.at[idx_v], rows_v, sem).wait()
            pltpu.sync_copy(rows_v, o1_hbm.at[pl.ds(base + off, CH)])

    return gk(table, idx0, idx1)


def _mm_body(te_ref, xs_ref, w_ref, b_ref, o_ref):
    acc = jax.lax.dot_general(
        xs_ref[...], w_ref[0], (((1,), (1,)), ((), ())),
        preferred_element_type=jnp.float32)
    o_ref[...] = acc + b_ref[0]


def _grouped_mm(xs, tile_expert, expert_w, expert_b):
    grid_spec = pltpu.PrefetchScalarGridSpec(
        num_scalar_prefetch=1,
        grid=(NT,),
        in_specs=[
            pl.BlockSpec((T, D), lambda i, te: (i, 0)),
            pl.BlockSpec((1, D, D), lambda i, te: (te[i], 0, 0)),
            pl.BlockSpec((1, 1, D), lambda i, te: (te[i], 0, 0)),
        ],
        out_specs=pl.BlockSpec((T, D), lambda i, te: (i, 0)),
    )
    return pl.pallas_call(
        _mm_body,
        grid_spec=grid_spec,
        out_shape=jax.ShapeDtypeStruct((R, D), jnp.float32),
    )(tile_expert, xs, expert_w, expert_b.reshape(E, 1, D))


def _comb_body(y0_ref, y1_ref, g_ref, o_ref):
    g = g_ref[...]
    o_ref[...] = y0_ref[...] * g[:, 0:1] + y1_ref[...] * g[:, 1:2]


def _combine(y0, y1, gates):
    return pl.pallas_call(
        _comb_body,
        grid=(N // TB,),
        in_specs=[
            pl.BlockSpec((TB, D), lambda i: (i, 0)),
            pl.BlockSpec((TB, D), lambda i: (i, 0)),
            pl.BlockSpec((TB, K), lambda i: (i, 0)),
        ],
        out_specs=pl.BlockSpec((TB, D), lambda i: (i, 0)),
        out_shape=jax.ShapeDtypeStruct((N, D), jnp.float32),
    )(y0, y1, gates)


def kernel(x, router_w, expert_w, expert_b):
    b, s, d = x.shape
    x_flat = x.reshape(N, D)
    gates, eidx = _router(x_flat, router_w)
    tile_expert, pos = _dispatch_indices(eidx)
    pos2 = pos.reshape(N, K)
    xs = _sc_dispatch(x_flat, pos2[:, 0], pos2[:, 1])
    ys = _grouped_mm(xs, tile_expert, expert_w, expert_b)
    y0, y1 = _sc_gather2(ys, pos2[:, 0], pos2[:, 1])
    out = _combine(y0, y1, gates)
    return out.reshape(b, s, D)
